# baseline (speedup 1.0000x reference)
import jax
import jax.numpy as jnp
from jax import lax
from jax.experimental import pallas as pl
from jax.experimental.pallas import tpu as pltpu
from jax.experimental.pallas import tpu_sc as plsc

X_MIN, X_MAX = -50.0, 50.0
Y_MIN, Y_MAX = -50.0, 50.0
BEV_W = 200
BEV_H = 200
BEV = BEV_W * BEV_H
TRASH = BEV
ACC = 40016

NC, NS, L = 2, 16, 16
NW = NC * NS

B = 2
ND = 354
HW = 704
C = 64
P = ND * HW
SLABS_PER_TILE = (ND + NW - 1) // NW
K = 2048
RPT = K // NS
NCHUNK = (P + K - 1) // K
P_PAD = NCHUNK * K
VECS = HW // L

_mesh = plsc.VectorSubcoreMesh(
    core_axis_name="c", subcore_axis_name="s", num_cores=NC, num_subcores=NS
)
_params = pltpu.CompilerParams(
    use_tc_tiling_on_sc=False, needs_layout_passes=False
)


def _phase1_body(coords_hbm, idx_hbm, cbuf, ibuf):
    w = lax.axis_index("s") * NC + lax.axis_index("c")
    lane = lax.iota(jnp.int32, L)
    trash = jnp.full((L,), TRASH, jnp.int32)
    for b in range(B):
        @pl.loop(0, SLABS_PER_TILE)
        def _slab(k):
            s = w + k * NW
            @pl.when(s < ND)
            def _():
                pltpu.sync_copy(coords_hbm.at[b, s], cbuf)

                @pl.loop(0, VECS)
                def _vec(j):
                    g = lane * (3 * VECS) + 3 * j
                    x = plsc.load_gather(cbuf, [g])
                    y = plsc.load_gather(cbuf, [g + 1])
                    xf = (x - X_MIN) * (BEV_W / (X_MAX - X_MIN))
                    yf = (y - Y_MIN) * (BEV_H / (Y_MAX - Y_MIN))
                    xf = jnp.minimum(jnp.maximum(xf, -2.0e9), 2.0e9)
                    yf = jnp.minimum(jnp.maximum(yf, -2.0e9), 2.0e9)
                    xi = xf.astype(jnp.int32)
                    yi = yf.astype(jnp.int32)
                    valid = (
                        (xi >= 0) & (xi < BEV_W) & (yi >= 0) & (yi < BEV_H)
                    )
                    lin = yi * BEV_W + xi
                    lin = jnp.where(valid, lin, TRASH)
                    ibuf[pl.ds(j * L, L)] = lin

                pltpu.sync_copy(ibuf, idx_hbm.at[b, pl.ds(s * HW, HW)])

        @pl.when(w == 0)
        def _():
            @pl.loop(0, (P_PAD - P) // L)
            def _pad(j):
                ibuf[pl.ds(j * L, L)] = trash

            pltpu.sync_copy(
                ibuf.at[pl.ds(0, P_PAD - P)], idx_hbm.at[b, pl.ds(P, P_PAD - P)]
            )


_phase1 = pl.kernel(
    _phase1_body,
    out_type=jax.ShapeDtypeStruct((B, P_PAD), jnp.int32),
    mesh=_mesh,
    compiler_params=_params,
    scratch_types=[
        pltpu.VMEM((3 * HW,), jnp.float32),
        pltpu.VMEM((HW,), jnp.int32),
    ],
)


def _phase2_body(
    idx_hbm, feats_hbm, out_hbm,
    acc0, acc1, fbuf, tbuf, sbuf, ibuf, staging, fsem, isem,
):
    co = lax.axis_index("c")
    sid = lax.axis_index("s")
    lane = lax.iota(jnp.int32, L)
    zeros = jnp.zeros((L,), jnp.float32)
    c0 = sid * 4 + co * 2
    c1 = c0 + 1
    for b in range(B):
        def fcopy(slot, k):
            return pltpu.make_async_copy(
                feats_hbm.at[b, pl.ds(k * K + sid * RPT, RPT), :],
                fbuf.at[slot],
                fsem.at[slot],
            )

        def icopy(slot, k):
            return pltpu.make_async_copy(
                idx_hbm.at[b, pl.ds(k * K, K)], ibuf.at[slot], isem.at[slot]
            )

        def fguard(k):
            return k * K + (sid + 1) * RPT <= P

        def issue(slot, k):
            @pl.when(fguard(k))
            def _():
                fcopy(slot, k).start()
            icopy(slot, k).start()

        @pl.loop(0, ACC // L, unroll=8)
        def _zero(i):
            acc0[pl.ds(i * L, L)] = zeros
            acc1[pl.ds(i * L, L)] = zeros

        issue(0, 0)

        @pl.loop(0, NCHUNK)
        def _chunk(k):
            slot = k & 1
            @pl.when(k + 1 < NCHUNK)
            def _():
                issue(1 - slot, k + 1)

            @pl.when(fguard(k))
            def _():
                fcopy(slot, k).wait()

            @pl.loop(0, 2 * NS)
            def _ch(l):
                cl = (l >> 1) * 4 + (l & 1) + co * 2
                colv = lane * 0 + cl

                @pl.loop(0, RPT // L)
                def _v(v):
                    rows = lane + v * L
                    val = plsc.load_gather(fbuf.at[slot], [rows, colv])
                    tbuf[l, pl.ds(v * L, L)] = val

            pltpu.sync_copy(
                tbuf, staging.at[slot, :, pl.ds(sid * RPT, RPT)]
            )
            plsc.subcore_barrier()
            pltpu.sync_copy(staging.at[slot, pl.ds(sid * 2, 2), :], sbuf)

            icopy(slot, k).wait()

            @pl.loop(0, K // L, unroll=4)
            def _vec(v):
                iv = ibuf[slot, pl.ds(v * L, L)]
                f0 = sbuf[0, pl.ds(v * L, L)]
                plsc.addupdate_scatter(acc0, [iv], f0)
                f1 = sbuf[1, pl.ds(v * L, L)]
                plsc.addupdate_scatter(acc1, [iv], f1)

        pltpu.sync_copy(acc0.at[pl.ds(0, BEV)], out_hbm.at[b, c0])
        pltpu.sync_copy(acc1.at[pl.ds(0, BEV)], out_hbm.at[b, c1])


_phase2 = pl.kernel(
    _phase2_body,
    out_type=jax.ShapeDtypeStruct((B, C, BEV), jnp.float32),
    mesh=_mesh,
    compiler_params=_params,
    scratch_types=[
        pltpu.VMEM((ACC,), jnp.float32),
        pltpu.VMEM((ACC,), jnp.float32),
        pltpu.VMEM((2, RPT, C), jnp.float32),
        pltpu.VMEM((2 * NS, RPT), jnp.float32),
        pltpu.VMEM((2, K), jnp.float32),
        pltpu.VMEM((2, K), jnp.int32),
        pltpu.VMEM_SHARED((2, 2 * NS, K), jnp.float32),
        pltpu.SemaphoreType.DMA((2,)),
        pltpu.SemaphoreType.DMA((2,)),
    ],
)


def kernel(coords_world, lifted_features):
    b, n, d, c, h, w = lifted_features.shape
    coords = coords_world.reshape(b, n * d, h * w * 3)
    feats = jnp.transpose(lifted_features, (0, 1, 2, 5, 4, 3)).reshape(
        b, n * d * h * w, c
    )
    idx = _phase1(coords)
    bev = _phase2(idx, feats)
    return bev.reshape(b, c, BEV_H, BEV_W)


# speedup vs baseline: 2.1674x; 2.1674x over previous
import jax
import jax.numpy as jnp
from jax import lax
from jax.experimental import pallas as pl
from jax.experimental.pallas import tpu as pltpu
from jax.experimental.pallas import tpu_sc as plsc

X_MIN, X_MAX = -50.0, 50.0
Y_MIN, Y_MAX = -50.0, 50.0
BEV_W = 200
BEV_H = 200
BEV = BEV_W * BEV_H
TRASH = BEV
ACC = 40016

NC, NS, L = 2, 16, 16
NW = NC * NS

B = 2
ND = 354
HW = 704
C = 64
CH = C // NC
P = ND * HW
SLABS_PER_TILE = (ND + NW - 1) // NW
VECS = HW // L

KT = 480
KC = KT * NS
NFULL = P // KC
KTAIL = (P - NFULL * KC) // NS
ZROWS = ACC // NS
WPIECE = 2000

_mesh = plsc.VectorSubcoreMesh(
    core_axis_name="c", subcore_axis_name="s", num_cores=NC, num_subcores=NS
)
_params = pltpu.CompilerParams(
    use_tc_tiling_on_sc=False, needs_layout_passes=False
)


def _phase1_body(coords_hbm, idx_hbm, cbuf, ibuf):
    w = lax.axis_index("s") * NC + lax.axis_index("c")
    lane = lax.iota(jnp.int32, L)
    for b in range(B):
        @pl.loop(0, SLABS_PER_TILE)
        def _slab(k):
            s = w + k * NW
            @pl.when(s < ND)
            def _():
                pltpu.sync_copy(coords_hbm.at[b, s], cbuf)

                @pl.loop(0, VECS)
                def _vec(j):
                    g = lane * (3 * VECS) + 3 * j
                    x = plsc.load_gather(cbuf, [g])
                    y = plsc.load_gather(cbuf, [g + 1])
                    xf = (x - X_MIN) * (BEV_W / (X_MAX - X_MIN))
                    yf = (y - Y_MIN) * (BEV_H / (Y_MAX - Y_MIN))
                    xf = jnp.minimum(jnp.maximum(xf, -2.0e9), 2.0e9)
                    yf = jnp.minimum(jnp.maximum(yf, -2.0e9), 2.0e9)
                    xi = xf.astype(jnp.int32)
                    yi = yf.astype(jnp.int32)
                    valid = (
                        (xi >= 0) & (xi < BEV_W) & (yi >= 0) & (yi < BEV_H)
                    )
                    lin = yi * BEV_W + xi
                    lin = jnp.where(valid, lin, TRASH)
                    ibuf[pl.ds(j * L, L)] = lin

                pltpu.sync_copy(ibuf, idx_hbm.at[b, pl.ds(s * HW, HW)])


_phase1 = pl.kernel(
    _phase1_body,
    out_type=jax.ShapeDtypeStruct((B, P), jnp.int32),
    mesh=_mesh,
    compiler_params=_params,
    scratch_types=[
        pltpu.VMEM((3 * HW,), jnp.float32),
        pltpu.VMEM((HW,), jnp.int32),
    ],
)


def _phase2_body(
    idx_hbm, feats_hbm, out_hbm,
    acc, fbuf, ibuf, cbuf, obuf, fsem, isem,
):
    co = lax.axis_index("c")
    sid = lax.axis_index("s")
    lane = lax.iota(jnp.int32, L)
    zeros = jnp.zeros((L,), jnp.float32)

    zpieces = []
    r0 = 0
    while r0 < ZROWS:
        zpieces.append((r0, min(KT, ZROWS - r0)))
        r0 += KT

    for b in range(B):
        @pl.loop(0, KT, unroll=8)
        def _zfill(i):
            fbuf[0, i, pl.ds(0, L)] = zeros
            fbuf[0, i, pl.ds(L, L)] = zeros

        zbase = sid * ZROWS
        for r0, rn in zpieces:
            pltpu.sync_copy(
                fbuf.at[0, pl.ds(0, rn), :],
                acc.at[pl.ds(zbase + r0, rn), :],
            )
        plsc.subcore_barrier()

        def fcopy(slot, k):
            return pltpu.make_async_copy(
                feats_hbm.at[
                    b, pl.ds(k * KC + sid * KT, KT), pl.ds(co * CH, CH)
                ],
                fbuf.at[slot],
                fsem.at[slot],
            )

        def icopy(slot, k):
            return pltpu.make_async_copy(
                idx_hbm.at[b, pl.ds(k * KC + sid * KT, KT)],
                ibuf.at[slot],
                isem.at[slot],
            )

        def issue(slot, k):
            fcopy(slot, k).start()
            icopy(slot, k).start()

        issue(0, 0)

        @pl.loop(0, NFULL)
        def _chunk(k):
            slot = k & 1
            @pl.when(k + 1 < NFULL)
            def _():
                issue(1 - slot, k + 1)

            fcopy(slot, k).wait()
            icopy(slot, k).wait()
            pltpu.sync_copy(fbuf.at[slot], acc.at[ibuf.at[slot]], add=True)

        tbase = NFULL * KC + sid * KTAIL
        pltpu.sync_copy(
            feats_hbm.at[b, pl.ds(tbase, KTAIL), pl.ds(co * CH, CH)],
            fbuf.at[0, pl.ds(0, KTAIL), :],
        )
        pltpu.sync_copy(
            idx_hbm.at[b, pl.ds(tbase, KTAIL)], ibuf.at[0, pl.ds(0, KTAIL)]
        )
        pltpu.sync_copy(
            fbuf.at[0, pl.ds(0, KTAIL), :],
            acc.at[ibuf.at[0, pl.ds(0, KTAIL)]],
            add=True,
        )

        plsc.subcore_barrier()

        for p in range(BEV // WPIECE):
            pltpu.sync_copy(
                acc.at[pl.ds(p * WPIECE, WPIECE), pl.ds(sid * 2, 2)], cbuf
            )
            for j in range(2):
                cg = co * CH + sid * 2 + j

                @pl.loop(0, WPIECE // L, unroll=4)
                def _g(v):
                    rows = v * L + lane
                    cols = lane * 0 + j
                    obuf[pl.ds(v * L, L)] = plsc.load_gather(
                        cbuf, [rows, cols]
                    )

                pltpu.sync_copy(
                    obuf, out_hbm.at[b, cg, pl.ds(p * WPIECE, WPIECE)]
                )
        plsc.subcore_barrier()


_phase2 = pl.kernel(
    _phase2_body,
    out_type=jax.ShapeDtypeStruct((B, C, BEV), jnp.float32),
    mesh=_mesh,
    compiler_params=_params,
    scratch_types=[
        pltpu.VMEM_SHARED((ACC, CH), jnp.float32),
        pltpu.VMEM((2, KT, CH), jnp.float32),
        pltpu.VMEM((2, KT), jnp.int32),
        pltpu.VMEM((WPIECE, 2), jnp.float32),
        pltpu.VMEM((WPIECE,), jnp.float32),
        pltpu.SemaphoreType.DMA((2,)),
        pltpu.SemaphoreType.DMA((2,)),
    ],
)


def kernel(coords_world, lifted_features):
    b, n, d, c, h, w = lifted_features.shape
    coords = coords_world.reshape(b, n * d, h * w * 3)
    feats = jnp.transpose(lifted_features, (0, 1, 2, 5, 4, 3)).reshape(
        b, n * d * h * w, c
    )
    idx = _phase1(coords)
    bev = _phase2(idx, feats)
    return bev.reshape(b, c, BEV_H, BEV_W)
